# baseline (speedup 1.0000x reference)
import jax
import jax.numpy as jnp
from jax import lax
from jax.experimental import pallas as pl
from jax.experimental.pallas import tpu as pltpu
from jax.experimental.pallas import tpu_sc as plsc

B, S, D, E = 4, 8192, 2048, 64
VEC = 16
NC, NS = 2, 16
NW = NC * NS
WPB = NW // B

S_TC = 6144
S_SC = S - S_TC
RPW = S_SC // WPB
R = 16
NCHUNK = RPW // R
NG_SC = NCHUNK // 2

RB = 256
CPB_TC = S_TC // RB
NCH_TC = B * CPB_TC
NBUF = 8
NG_TC = NCH_TC // NBUF


def _sc_sum(x_hbm, out_hbm, buf0, buf1, acc, sem0, sem1):
    wid = lax.axis_index("s") * NC + lax.axis_index("c")
    batch = wid // WPB
    slot = wid % WPB
    base = batch * S + S_TC + slot * RPW

    def start(c, buf, sem):
        pltpu.async_copy(x_hbm.at[pl.ds(base + c * R, R)], buf, sem)

    def wait(buf, sem):
        pltpu.make_async_copy(x_hbm.at[pl.ds(0, R)], buf, sem).wait()

    @plsc.parallel_loop(0, D // VEC)
    def _zero(j):
        acc[0, pl.ds(j * VEC, VEC)] = jnp.zeros((VEC,), jnp.float32)

    start(0, buf0, sem0)
    start(1, buf1, sem1)

    def accum(buf):
        @plsc.parallel_loop(0, D // VEC, unroll=2)
        def _strip(j):
            col = j * VEC
            v = buf[0, pl.ds(col, VEC)]
            for r in range(1, R):
                v = v + buf[r, pl.ds(col, VEC)]
            acc[0, pl.ds(col, VEC)] += v

    def body(g, _):
        wait(buf0, sem0)
        accum(buf0)

        @pl.when(g < NG_SC - 1)
        def _p0():
            start(2 * g + 2, buf0, sem0)

        wait(buf1, sem1)
        accum(buf1)

        @pl.when(g < NG_SC - 1)
        def _p1():
            start(2 * g + 3, buf1, sem1)

        return 0

    lax.fori_loop(0, NG_SC, body, 0)
    pltpu.sync_copy(acc, out_hbm.at[pl.ds(wid, 1)])


_sc_sum_call = pl.kernel(
    _sc_sum,
    out_type=jax.ShapeDtypeStruct((NW, D), jnp.float32),
    mesh=plsc.VectorSubcoreMesh(core_axis_name="c", subcore_axis_name="s"),
    scratch_types=[
        pltpu.VMEM((R, D), jnp.float32),
        pltpu.VMEM((R, D), jnp.float32),
        pltpu.VMEM((1, D), jnp.float32),
        pltpu.SemaphoreType.DMA,
        pltpu.SemaphoreType.DMA,
    ],
)


def _tc_kernel(x_hbm, out_ref, *rest):
    bufs = list(rest[:NBUF])
    acc_ref = rest[NBUF]
    sems = list(rest[NBUF + 1:])

    def row_of(c):
        return (c // CPB_TC) * S + (c % CPB_TC) * RB

    def start(c, k):
        pltpu.make_async_copy(
            x_hbm.at[pl.ds(row_of(c) * 1, RB)], bufs[k], sems[k]).start()

    def wait(k):
        pltpu.make_async_copy(
            x_hbm.at[pl.ds(0, RB)], bufs[k], sems[k]).wait()

    acc_ref[...] = jnp.zeros_like(acc_ref)
    for k in range(NBUF):
        start(k, k)

    def loop(g, _):
        for k in range(NBUF):
            c = NBUF * g + k
            wait(k)
            part = jnp.sum(bufs[k][...].reshape(RB // 8, 8, D), axis=0)
            batch = c // CPB_TC
            acc_ref[pl.ds(batch * 8, 8), :] += part

            @pl.when(c + NBUF < NCH_TC)
            def _next():
                start(c + NBUF, k)

        return 0

    lax.fori_loop(0, NG_TC, loop, 0)
    out_ref[...] = jnp.sum(
        acc_ref[...].reshape(B, 8, D), axis=1)


def _combine_kernel(t_ref, p_ref, w_ref, b_ref, out_ref):
    s = t_ref[...] + jnp.sum(p_ref[...], axis=1)
    logits = jax.lax.dot_general(
        s, w_ref[...],
        dimension_numbers=(((1,), (1,)), ((), ())),
        preferred_element_type=jnp.float32,
    ) + b_ref[...]
    out_ref[...] = jnp.argmax(logits, axis=1).astype(jnp.int32)[None, :]


def kernel(x, W, b):
    xf = x.reshape(B * S, D)
    sc_partials = _sc_sum_call(xf)
    tc_partial = pl.pallas_call(
        _tc_kernel,
        in_specs=[pl.BlockSpec(memory_space=pltpu.MemorySpace.HBM)],
        out_specs=pl.BlockSpec(memory_space=pltpu.MemorySpace.VMEM),
        out_shape=jax.ShapeDtypeStruct((B, D), jnp.float32),
        scratch_shapes=(
            [pltpu.VMEM((RB, D), jnp.float32)] * NBUF
            + [pltpu.VMEM((B * 8, D), jnp.float32)]
            + [pltpu.SemaphoreType.DMA] * NBUF
        ),
    )(xf)
    out = pl.pallas_call(
        _combine_kernel,
        in_specs=[
            pl.BlockSpec((B, D), lambda: (0, 0)),
            pl.BlockSpec((B, WPB, D), lambda: (0, 0, 0)),
            pl.BlockSpec((E, D), lambda: (0, 0)),
            pl.BlockSpec((1, E), lambda: (0, 0)),
        ],
        out_specs=pl.BlockSpec((1, B), lambda: (0, 0)),
        out_shape=jax.ShapeDtypeStruct((1, B), jnp.int32),
    )(tc_partial, sc_partials.reshape(B, WPB, D), W, b.reshape(1, E))
    return out.reshape(B)


# speedup vs baseline: 1.2268x; 1.2268x over previous
import jax
import jax.numpy as jnp
from jax import lax
from jax.experimental import pallas as pl
from jax.experimental.pallas import tpu as pltpu

B, S, D, E = 4, 8192, 2048, 64
RB = 512
NCH = B * S // RB
CPB = S // RB
NBUF = 4
NG = NCH // NBUF


def _tc_kernel(x_hbm, w_ref, b_ref, out_ref, *rest):
    bufs = list(rest[:NBUF])
    acc_ref = rest[NBUF]
    sems = list(rest[NBUF + 1:])

    def start(c, k):
        pltpu.make_async_copy(
            x_hbm.at[pl.ds(c * RB, RB)], bufs[k], sems[k]).start()

    def wait(k):
        pltpu.make_async_copy(
            x_hbm.at[pl.ds(0, RB)], bufs[k], sems[k]).wait()

    acc_ref[...] = jnp.zeros_like(acc_ref)
    for k in range(NBUF):
        start(k, k)

    def loop(g, _):
        for k in range(NBUF):
            c = NBUF * g + k
            wait(k)
            part = jnp.sum(bufs[k][...].reshape(RB // 8, 8, D), axis=0)
            batch = c // CPB
            acc_ref[pl.ds(batch * 8, 8), :] += part

            @pl.when(c + NBUF < NCH)
            def _next():
                start(c + NBUF, k)

        return 0

    lax.fori_loop(0, NG, loop, 0)

    s = jnp.sum(acc_ref[...].reshape(B, 8, D), axis=1)
    logits = jax.lax.dot_general(
        s, w_ref[...],
        dimension_numbers=(((1,), (1,)), ((), ())),
        preferred_element_type=jnp.float32,
    ) + b_ref[...]
    out_ref[...] = jnp.argmax(logits, axis=1).astype(jnp.int32)[None, :]


def kernel(x, W, b):
    out = pl.pallas_call(
        _tc_kernel,
        in_specs=[
            pl.BlockSpec(memory_space=pltpu.MemorySpace.HBM),
            pl.BlockSpec(memory_space=pltpu.MemorySpace.VMEM),
            pl.BlockSpec(memory_space=pltpu.MemorySpace.VMEM),
        ],
        out_specs=pl.BlockSpec(memory_space=pltpu.MemorySpace.VMEM),
        out_shape=jax.ShapeDtypeStruct((1, B), jnp.int32),
        scratch_shapes=(
            [pltpu.VMEM((RB, D), jnp.float32)] * NBUF
            + [pltpu.VMEM((B * 8, D), jnp.float32)]
            + [pltpu.SemaphoreType.DMA] * NBUF
        ),
    )(x.reshape(B * S, D), W, b.reshape(1, E))
    return out.reshape(B)


# speedup vs baseline: 1.2330x; 1.0051x over previous
import jax
import jax.numpy as jnp
from jax import lax
from jax.experimental import pallas as pl
from jax.experimental.pallas import tpu as pltpu

B, S, D, E = 4, 8192, 2048, 64
RB = 256
NCH = B * S // RB
CPB = S // RB
NBUF = 4
NG = NCH // NBUF


def _tc_kernel(x_hbm, w_ref, b_ref, out_ref, b0, b1, b2, b3, acc_ref,
               s0, s1, s2, s3):
    bufs = [b0, b1, b2, b3]
    sems = [s0, s1, s2, s3]

    def start(c, k):
        pltpu.make_async_copy(
            x_hbm.at[pl.ds(c * RB, RB)], bufs[k], sems[k]).start()

    def wait(k):
        pltpu.make_async_copy(
            x_hbm.at[pl.ds(0, RB)], bufs[k], sems[k]).wait()

    acc_ref[...] = jnp.zeros_like(acc_ref)
    for k in range(NBUF):
        start(k, k)

    def loop(g, _):
        for k in range(NBUF):
            c = NBUF * g + k
            wait(k)
            part = jnp.sum(bufs[k][...].reshape(RB // 8, 8, D), axis=0)
            batch = c // CPB
            acc_ref[pl.ds(batch * 8, 8), :] += part

            @pl.when(c + NBUF < NCH)
            def _next():
                start(c + NBUF, k)

        return 0

    lax.fori_loop(0, NG, loop, 0)

    s = jnp.sum(acc_ref[...].reshape(B, 8, D), axis=1)
    logits = jax.lax.dot_general(
        s, w_ref[...],
        dimension_numbers=(((1,), (1,)), ((), ())),
        preferred_element_type=jnp.float32,
    ) + b_ref[...]
    out_ref[...] = jnp.argmax(logits, axis=1).astype(jnp.int32)[None, :]


def kernel(x, W, b):
    out = pl.pallas_call(
        _tc_kernel,
        in_specs=[
            pl.BlockSpec(memory_space=pltpu.MemorySpace.HBM),
            pl.BlockSpec(memory_space=pltpu.MemorySpace.VMEM),
            pl.BlockSpec(memory_space=pltpu.MemorySpace.VMEM),
        ],
        out_specs=pl.BlockSpec(memory_space=pltpu.MemorySpace.VMEM),
        out_shape=jax.ShapeDtypeStruct((1, B), jnp.int32),
        scratch_shapes=[
            pltpu.VMEM((RB, D), jnp.float32),
            pltpu.VMEM((RB, D), jnp.float32),
            pltpu.VMEM((RB, D), jnp.float32),
            pltpu.VMEM((RB, D), jnp.float32),
            pltpu.VMEM((B * 8, D), jnp.float32),
            pltpu.SemaphoreType.DMA,
            pltpu.SemaphoreType.DMA,
            pltpu.SemaphoreType.DMA,
            pltpu.SemaphoreType.DMA,
        ],
    )(x.reshape(B * S, D), W, b.reshape(1, E))
    return out.reshape(B)
